# baseline (speedup 1.0000x reference)
import jax
import jax.numpy as jnp
from jax import lax
from jax.experimental import pallas as pl
from jax.experimental.pallas import tpu as pltpu

_SIZE_REG = 1.0
_ENT_REG = 1.0
_EPS = 1e-15


_SR = 8


def _stream_kernel(m_ref, f_ref, c_ref, s_ref, e_ref):
    m = m_ref[...].reshape(f_ref.shape)
    f_ref[...] = m
    c_ref[...] = 1.0 - m
    nseg = f_ref.shape[0] // _SR

    def ebody(k, carry):
        s_a, e_a = carry
        seg = f_ref[pl.ds(k * _SR, _SR), :]
        ent = (-seg * jnp.log(seg + _EPS)
               - (1.0 - seg) * jnp.log(1.0 - seg + _EPS))
        return s_a + jnp.sum(seg), e_a + jnp.sum(ent)

    s_a, e_a = lax.fori_loop(0, nseg, ebody, (jnp.float32(0), jnp.float32(0)))
    s_ref[...] = s_a.reshape(1, 1, 1)
    e_ref[...] = e_a.reshape(1, 1, 1)


def kernel(edge_filter, mask):
    B, E = edge_filter.shape
    n = B * E
    RB = 128
    G = B // RB
    f, c, s, e = pl.pallas_call(
        _stream_kernel,
        grid=(G,),
        in_specs=[pl.BlockSpec((RB * E,), lambda i: (i,))],
        out_specs=[
            pl.BlockSpec((RB, E), lambda i: (i, 0)),
            pl.BlockSpec((RB, E), lambda i: (i, 0)),
            pl.BlockSpec((1, 1, 1), lambda i: (i, 0, 0)),
            pl.BlockSpec((1, 1, 1), lambda i: (i, 0, 0)),
        ],
        out_shape=[
            jax.ShapeDtypeStruct((B, E), mask.dtype),
            jax.ShapeDtypeStruct((B, E), mask.dtype),
            jax.ShapeDtypeStruct((G, 1, 1), jnp.float32),
            jax.ShapeDtypeStruct((G, 1, 1), jnp.float32),
        ],
        compiler_params=pltpu.CompilerParams(
            dimension_semantics=("parallel",),
        ),
    )(mask)
    inv_n = 1.0 / n
    size_loss = jnp.sum(s) * (_SIZE_REG * inv_n)
    ent_loss = jnp.sum(e) * (_ENT_REG * inv_n)
    return f, c, size_loss, ent_loss


# speedup vs baseline: 1.4812x; 1.4812x over previous
import jax
import jax.numpy as jnp
from jax import lax
from jax.experimental import pallas as pl
from jax.experimental.pallas import tpu as pltpu

_SIZE_REG = 1.0
_ENT_REG = 1.0
_EPS = 1e-15


_SR = 8


def _stream_kernel(m_ref, f_ref, c_ref, s_ref, e_ref):
    f_ref[...] = m_ref[...].reshape(f_ref.shape)
    nseg = f_ref.shape[0] // _SR
    s_a = jnp.float32(0)
    e_a = jnp.float32(0)
    for k in range(nseg):
        seg = f_ref[k * _SR:(k + 1) * _SR, :]
        c_ref[k * _SR:(k + 1) * _SR, :] = 1.0 - seg
        ent = (-seg * jnp.log(seg + _EPS)
               - (1.0 - seg) * jnp.log(1.0 - seg + _EPS))
        s_a = s_a + jnp.sum(seg)
        e_a = e_a + jnp.sum(ent)
    s_ref[...] = s_a.reshape(1, 1, 1)
    e_ref[...] = e_a.reshape(1, 1, 1)


def kernel(edge_filter, mask):
    B, E = edge_filter.shape
    n = B * E
    RB = 256
    G = B // RB
    f, c, s, e = pl.pallas_call(
        _stream_kernel,
        grid=(G,),
        in_specs=[pl.BlockSpec((RB * E,), lambda i: (i,))],
        out_specs=[
            pl.BlockSpec((RB, E), lambda i: (i, 0)),
            pl.BlockSpec((RB, E), lambda i: (i, 0)),
            pl.BlockSpec((1, 1, 1), lambda i: (i, 0, 0)),
            pl.BlockSpec((1, 1, 1), lambda i: (i, 0, 0)),
        ],
        out_shape=[
            jax.ShapeDtypeStruct((B, E), mask.dtype),
            jax.ShapeDtypeStruct((B, E), mask.dtype),
            jax.ShapeDtypeStruct((G, 1, 1), jnp.float32),
            jax.ShapeDtypeStruct((G, 1, 1), jnp.float32),
        ],
        compiler_params=pltpu.CompilerParams(
            dimension_semantics=("parallel",),
        ),
    )(mask)
    inv_n = 1.0 / n
    size_loss = jnp.sum(s) * (_SIZE_REG * inv_n)
    ent_loss = jnp.sum(e) * (_ENT_REG * inv_n)
    return f, c, size_loss, ent_loss
